# Initial kernel scaffold; baseline (speedup 1.0000x reference)
import jax
import jax.numpy as jnp
from jax.experimental import pallas as pl


def kernel(x, edge_index, W1, b1, W_mu, b_mu, W_lv, b_lv):
    raise NotImplementedError("write your pallas kernel here")


# speedup vs baseline: 16.7997x; 16.7997x over previous
import functools

import jax
import jax.numpy as jnp
from jax import lax
from jax.experimental import pallas as pl
from jax.experimental.pallas import tpu as pltpu
from jax.experimental.pallas import tpu_sc as plsc

N = 10000
D = 128
E = 320000
NC, NS = 2, 16
NW = NC * NS
HD = D // NC
K = 128
NB = 80
NBT = NC * NB
NA = N + 8
ZR = 208
RPS = 624


@functools.cache
def _sc_mesh():
    return plsc.VectorSubcoreMesh(
        core_axis_name="c", subcore_axis_name="s", num_cores=NC, num_subcores=NS)


def _zero_rows(ref, nrows, width):
    z = jnp.zeros((16,), jnp.float32)

    def body(i, carry):
        for k in range(width // 16):
            ref[i, pl.ds(16 * k, 16)] = z
        return carry

    lax.fori_loop(0, nrows, body, 0)


def _zero_acc(s, zbuf, acc):
    r0 = s * RPS
    for t in range(3):
        pltpu.sync_copy(zbuf, acc.at[pl.ds(r0 + t * ZR, ZR)])

    @pl.when(s == NS - 1)
    def _():
        pltpu.sync_copy(zbuf.at[pl.ds(0, NA - NS * RPS)],
                        acc.at[pl.ds(NS * RPS, NA - NS * RPS)])


def _copy_out(c, s, acc, out_hbm):
    r0 = s * RPS
    pltpu.sync_copy(acc.at[pl.ds(r0, RPS)], out_hbm.at[c, pl.ds(r0, RPS)])

    @pl.when(s == NS - 1)
    def _():
        pltpu.sync_copy(acc.at[pl.ds(NS * RPS, N - NS * RPS)],
                        out_hbm.at[c, pl.ds(NS * RPS, N - NS * RPS)])


def _deg_body(dst_hbm, out_hbm, didx, onesb, zed, accd):
    c = lax.axis_index("c")
    s = lax.axis_index("s")
    g = c * NS + s

    one = jnp.ones((16,), jnp.float32)

    def fill(i, carry):
        onesb[i, :] = one
        return carry

    lax.fori_loop(0, K, fill, 0)
    _zero_rows(zed, ZR, 16)
    _zero_acc(s, zed, accd)
    plsc.subcore_barrier()

    pltpu.sync_copy(dst_hbm.at[pl.ds(g * NB, NB)], didx)

    def body(i, carry):
        pltpu.sync_copy(onesb, accd.at[didx.at[i]], add=True)
        return carry

    lax.fori_loop(0, NB, body, 0)

    plsc.subcore_barrier()
    _copy_out(c, s, accd, out_hbm)


@functools.cache
def _deg_call():
    return pl.kernel(
        _deg_body,
        out_type=jax.ShapeDtypeStruct((NC, N, 16), jnp.float32),
        mesh=_sc_mesh(),
        compiler_params=pltpu.CompilerParams(use_tc_tiling_on_sc=False),
        scratch_types=[
            pltpu.VMEM((NB, K), jnp.int32),
            pltpu.VMEM((K, 16), jnp.float32),
            pltpu.VMEM((ZR, 16), jnp.float32),
            pltpu.VMEM_SHARED((NA, 16), jnp.float32),
        ],
    )


def _prop_body(y_hbm, src_hbm, dst_hbm, out_hbm,
               sidx, didx, buf0, buf1, ze, acc, sem0, sem1):
    c = lax.axis_index("c")
    s = lax.axis_index("s")

    _zero_rows(ze, ZR, HD)
    _zero_acc(s, ze, acc)
    plsc.subcore_barrier()

    pltpu.sync_copy(src_hbm.at[pl.ds(s * NBT, NBT)], sidx)
    pltpu.sync_copy(dst_hbm.at[pl.ds(s * NBT, NBT)], didx)

    yc = y_hbm.at[c]

    def g_start(i, buf, sem):
        pltpu.async_copy(yc.at[sidx.at[i]], buf, sem)

    def g_wait(buf, sem):
        pltpu.make_async_copy(yc.at[sidx.at[0]], buf, sem).wait()

    def s_add(i, buf):
        pltpu.sync_copy(buf, acc.at[didx.at[i]], add=True)

    g_start(0, buf0, sem0)

    def body(j, carry):
        a = 2 * j
        g_start(a + 1, buf1, sem1)
        g_wait(buf0, sem0)
        s_add(a, buf0)
        g_start(a + 2, buf0, sem0)
        g_wait(buf1, sem1)
        s_add(a + 1, buf1)
        return carry

    lax.fori_loop(0, NBT // 2 - 1, body, 0)

    g_start(NBT - 1, buf1, sem1)
    g_wait(buf0, sem0)
    s_add(NBT - 2, buf0)
    g_wait(buf1, sem1)
    s_add(NBT - 1, buf1)

    plsc.subcore_barrier()
    _copy_out(c, s, acc, out_hbm)


@functools.cache
def _prop_call():
    return pl.kernel(
        _prop_body,
        out_type=jax.ShapeDtypeStruct((NC, N, HD), jnp.float32),
        mesh=_sc_mesh(),
        compiler_params=pltpu.CompilerParams(use_tc_tiling_on_sc=False),
        scratch_types=[
            pltpu.VMEM((NBT, K), jnp.int32),
            pltpu.VMEM((NBT, K), jnp.int32),
            pltpu.VMEM((K, HD), jnp.float32),
            pltpu.VMEM((K, HD), jnp.float32),
            pltpu.VMEM((ZR, HD), jnp.float32),
            pltpu.VMEM_SHARED((NA, HD), jnp.float32),
            pltpu.SemaphoreType.DMA,
            pltpu.SemaphoreType.DMA,
        ],
    )


_BR = 1000
_GRID = N // _BR


def _tc1_body(degp_ref, x_ref, w1_ref, y1_ref, dinv_ref):
    deg16 = degp_ref[0] + degp_ref[1] + 1.0
    dinv16 = lax.rsqrt(deg16)
    dinv = dinv16[:, 0:1]
    dinv_ref[...] = dinv
    y1 = jnp.dot(x_ref[...], w1_ref[...],
                 preferred_element_type=jnp.float32) * dinv
    y1_ref[0] = y1[:, :HD]
    y1_ref[1] = y1[:, HD:]


_tc1_call = pl.pallas_call(
    _tc1_body,
    grid=(_GRID,),
    in_specs=[
        pl.BlockSpec((NC, _BR, 16), lambda i: (0, i, 0)),
        pl.BlockSpec((_BR, D), lambda i: (i, 0)),
        pl.BlockSpec((D, D), lambda i: (0, 0)),
    ],
    out_specs=[
        pl.BlockSpec((NC, _BR, HD), lambda i: (0, i, 0)),
        pl.BlockSpec((_BR, 1), lambda i: (i, 0)),
    ],
    out_shape=[
        jax.ShapeDtypeStruct((NC, N, HD), jnp.float32),
        jax.ShapeDtypeStruct((N, 1), jnp.float32),
    ],
)


def _tc2_body(s_ref, y1_ref, dinv_ref, b1_ref, y2_ref):
    dinv = dinv_ref[...]
    for half in range(NC):
        p = (s_ref[half] + y1_ref[half]) * dinv + b1_ref[0, :, HD * half:HD * (half + 1)]
        y2_ref[half] = jnp.maximum(p, 0.0) * dinv


_tc2_call = pl.pallas_call(
    _tc2_body,
    grid=(_GRID,),
    in_specs=[
        pl.BlockSpec((NC, _BR, HD), lambda i: (0, i, 0)),
        pl.BlockSpec((NC, _BR, HD), lambda i: (0, i, 0)),
        pl.BlockSpec((_BR, 1), lambda i: (i, 0)),
        pl.BlockSpec((1, 1, D), lambda i: (0, 0, 0)),
    ],
    out_specs=pl.BlockSpec((NC, _BR, HD), lambda i: (0, i, 0)),
    out_shape=jax.ShapeDtypeStruct((NC, N, HD), jnp.float32),
)


def _tc3_body(s_ref, y2_ref, dinv_ref, wmu_ref, wlv_ref, bmu_ref, blv_ref,
              zmu_ref, zlv_ref):
    dinv = dinv_ref[...]
    p2 = jnp.concatenate([s_ref[0] + y2_ref[0], s_ref[1] + y2_ref[1]],
                         axis=1) * dinv
    zmu_ref[...] = jnp.dot(p2, wmu_ref[...],
                           preferred_element_type=jnp.float32) + bmu_ref[...]
    zlv_ref[...] = jnp.dot(p2, wlv_ref[...],
                           preferred_element_type=jnp.float32) + blv_ref[...]


_tc3_call = pl.pallas_call(
    _tc3_body,
    grid=(_GRID,),
    in_specs=[
        pl.BlockSpec((NC, _BR, HD), lambda i: (0, i, 0)),
        pl.BlockSpec((NC, _BR, HD), lambda i: (0, i, 0)),
        pl.BlockSpec((_BR, 1), lambda i: (i, 0)),
        pl.BlockSpec((D, D), lambda i: (0, 0)),
        pl.BlockSpec((D, D), lambda i: (0, 0)),
        pl.BlockSpec((1, D), lambda i: (0, 0)),
        pl.BlockSpec((1, D), lambda i: (0, 0)),
    ],
    out_specs=[
        pl.BlockSpec((_BR, D), lambda i: (i, 0)),
        pl.BlockSpec((_BR, D), lambda i: (i, 0)),
    ],
    out_shape=[
        jax.ShapeDtypeStruct((N, D), jnp.float32),
        jax.ShapeDtypeStruct((N, D), jnp.float32),
    ],
)


def kernel(x, edge_index, W1, b1, W_mu, b_mu, W_lv, b_lv):
    ei = edge_index.astype(jnp.int32)
    pad = NW * NB * K - E
    src2 = jnp.concatenate([ei[0], jnp.zeros((pad,), jnp.int32)]).reshape(NW * NB, K)
    dst2 = jnp.concatenate([ei[1], jnp.full((pad,), N, jnp.int32)]).reshape(NW * NB, K)

    degp = _deg_call()(dst2)
    y1, dinv = _tc1_call(degp, x, W1)
    s1 = _prop_call()(y1, src2, dst2)
    y2 = _tc2_call(s1, y1, dinv, b1.reshape(1, 1, D))
    s2 = _prop_call()(y2, src2, dst2)
    z_mu, z_lv = _tc3_call(s2, y2, dinv, W_mu, W_lv,
                           b_mu.reshape(1, D), b_lv.reshape(1, D))
    return (z_mu, z_lv)
